# baseline (speedup 1.0000x reference)
import functools

import jax
import jax.numpy as jnp
from jax import lax
from jax.experimental import pallas as pl
from jax.experimental.pallas import tpu as pltpu
from jax.experimental.pallas import tpu_sc as plsc

N = 10000
E = 320000
D = 128
G = 64
NW = 32
EPW = E // NW
CH = 128
EPWP = 10240
NCH = EPWP // CH
EPAD = NW * EPWP - E
NPAD = 10240
RPT = NPAD // 16
NB = 10
BR = N // NB


def _degree_body(comb, deg_out, idx_v, ones_v, zero_v, deg_sh, ssem):
    cid = lax.axis_index("c")
    sid = lax.axis_index("s")
    wid = sid * 2 + cid
    for i in range(8):
        ones_v[pl.ds(i * 16, 16)] = jnp.full((16,), 1.0, jnp.float32)
    for i in range(40):
        zero_v[pl.ds(i * 16, 16)] = jnp.zeros((16,), jnp.float32)
    pltpu.sync_copy(zero_v, deg_sh.at[pl.ds(sid * 640, 640)])
    pltpu.sync_copy(comb.at[wid], idx_v)
    plsc.subcore_barrier()

    def step(j, carry):
        @pl.when(j >= 8)
        def _drain():
            pltpu.make_async_copy(ones_v, deg_sh.at[idx_v.at[j - 8, 1]],
                                  ssem.at[lax.rem(j, 8)]).wait()

        pltpu.async_copy(ones_v, deg_sh.at[idx_v.at[j, 1]],
                         ssem.at[lax.rem(j, 8)], add=True)
        return carry

    lax.fori_loop(0, NCH, step, 0)
    for t in range(8):
        pltpu.make_async_copy(ones_v, deg_sh.at[idx_v.at[NCH - 8 + t, 1]],
                              ssem.at[(NCH - 8 + t) % 8]).wait()
    plsc.subcore_barrier()
    pltpu.sync_copy(deg_sh.at[pl.ds(sid * 640, 640)],
                    deg_out.at[cid, pl.ds(sid * 640, 640)])


_degree = functools.partial(
    pl.kernel,
    out_type=jax.ShapeDtypeStruct((2, NPAD), jnp.float32),
    mesh=plsc.VectorSubcoreMesh(core_axis_name="c", subcore_axis_name="s"),
    scratch_types=[
        pltpu.VMEM((NCH, 2, CH), jnp.int32),
        pltpu.VMEM((CH,), jnp.float32),
        pltpu.VMEM((640,), jnp.float32),
        pltpu.VMEM_SHARED((NPAD,), jnp.float32),
        pltpu.SemaphoreType.DMA((8,)),
    ],
)(_degree_body)


def _scatter_body(hs, comb, out, idx_v, rows_v, acc_sh, gsem, isem, ssem,
                  fsem):
    cid = lax.axis_index("c")
    sid = lax.axis_index("s")
    wid = sid * 2 + cid

    def zrow(r, carry):
        for c in range(8):
            rows_v[0, r, pl.ds(c * 16, 16)] = jnp.zeros((16,), jnp.float32)
        return carry

    lax.fori_loop(0, CH, zrow, 0)
    for k in range(5):
        pltpu.async_copy(rows_v.at[0],
                         acc_sh.at[pl.ds(sid * RPT + k * CH, CH)], fsem.at[k])
    pltpu.sync_copy(comb.at[wid, 0], idx_v.at[0])
    for k in range(5):
        pltpu.make_async_copy(rows_v.at[0],
                              acc_sh.at[pl.ds(sid * RPT + k * CH, CH)],
                              fsem.at[k]).wait()
    plsc.subcore_barrier()
    pltpu.async_copy(hs.at[idx_v.at[0, 0]], rows_v.at[0], gsem.at[0])
    pltpu.async_copy(comb.at[wid, 1], idx_v.at[1], isem.at[1])

    def step(j, carry):
        p = lax.rem(j, 2)
        pn = 1 - p
        q = lax.rem(j, 3)
        qn = lax.rem(j + 1, 3)

        @pl.when((j + 1 < NCH) & (j >= 1))
        def _row_free():
            pltpu.make_async_copy(rows_v.at[pn],
                                  acc_sh.at[idx_v.at[lax.rem(j + 2, 3), 1]],
                                  ssem.at[pn]).wait()

        @pl.when(j + 1 < NCH)
        def _next_gather():
            pltpu.make_async_copy(comb.at[wid, j + 1], idx_v.at[qn],
                                  isem.at[qn]).wait()
            pltpu.async_copy(hs.at[idx_v.at[qn, 0]], rows_v.at[pn],
                             gsem.at[pn])

        pltpu.make_async_copy(hs.at[idx_v.at[q, 0]], rows_v.at[p],
                              gsem.at[p]).wait()
        pltpu.async_copy(rows_v.at[p], acc_sh.at[idx_v.at[q, 1]],
                         ssem.at[p], add=True)

        @pl.when(j + 2 < NCH)
        def _next_idx():
            pltpu.async_copy(comb.at[wid, j + 2], idx_v.at[lax.rem(j + 2, 3)],
                             isem.at[lax.rem(j + 2, 3)])

        return carry

    lax.fori_loop(0, NCH, step, 0)
    pltpu.make_async_copy(rows_v.at[(NCH - 2) % 2],
                          acc_sh.at[idx_v.at[(NCH - 2) % 3, 1]],
                          ssem.at[(NCH - 2) % 2]).wait()
    pltpu.make_async_copy(rows_v.at[(NCH - 1) % 2],
                          acc_sh.at[idx_v.at[(NCH - 1) % 3, 1]],
                          ssem.at[(NCH - 1) % 2]).wait()
    plsc.subcore_barrier()
    for k in range(5):
        r0 = sid * RPT + k * 128
        pltpu.async_copy(acc_sh.at[pl.ds(r0, 128)],
                         out.at[cid, pl.ds(r0, 128)], fsem.at[k])
    for k in range(5):
        r0 = sid * RPT + k * 128
        pltpu.make_async_copy(acc_sh.at[pl.ds(r0, 128)],
                              out.at[cid, pl.ds(r0, 128)], fsem.at[k]).wait()


_scatter = functools.partial(
    pl.kernel,
    out_type=jax.ShapeDtypeStruct((2, NPAD, D), jnp.float32),
    mesh=plsc.VectorSubcoreMesh(core_axis_name="c", subcore_axis_name="s"),
    scratch_types=[
        pltpu.VMEM((3, 2, CH), jnp.int32),
        pltpu.VMEM((2, CH, D), jnp.float32),
        pltpu.VMEM_SHARED((NPAD, D), jnp.float32),
        pltpu.SemaphoreType.DMA((2,)),
        pltpu.SemaphoreType.DMA((3,)),
        pltpu.SemaphoreType.DMA((2,)),
        pltpu.SemaphoreType.DMA((5,)),
    ],
)(_scatter_body)


def _tc_a_body(x_ref, w1_ref, deg_ref, hs_ref):
    dgp = deg_ref[...]
    dinv = lax.rsqrt(dgp[0] + dgp[1] + 1.0)
    hs_ref[...] = jnp.dot(x_ref[...], w1_ref[...],
                          preferred_element_type=jnp.float32) * dinv


def _tc_a(x, w1, deg3):
    return pl.pallas_call(
        _tc_a_body,
        grid=(NB,),
        in_specs=[
            pl.BlockSpec((BR, D), lambda i: (i, 0)),
            pl.BlockSpec((D, D), lambda i: (0, 0)),
            pl.BlockSpec((2, BR, 1), lambda i: (0, i, 0)),
        ],
        out_specs=pl.BlockSpec((BR, D), lambda i: (i, 0)),
        out_shape=jax.ShapeDtypeStruct((N, D), jnp.float32),
    )(x, w1, deg3)


def _tc_b_body(s1_ref, hs1_ref, deg_ref, b1_ref, w2_ref, hs2_ref):
    dgp = deg_ref[...]
    dinv = lax.rsqrt(dgp[0] + dgp[1] + 1.0)
    s = s1_ref[...]
    o1 = (s[0] + s[1] + hs1_ref[...]) * dinv + b1_ref[...]
    r = jnp.maximum(o1, 0.0)
    hs2_ref[...] = jnp.dot(r, w2_ref[...],
                           preferred_element_type=jnp.float32) * dinv


def _tc_b(s1, hs1, deg3, b1, w2):
    return pl.pallas_call(
        _tc_b_body,
        grid=(NB,),
        in_specs=[
            pl.BlockSpec((2, BR, D), lambda i: (0, i, 0)),
            pl.BlockSpec((BR, D), lambda i: (i, 0)),
            pl.BlockSpec((2, BR, 1), lambda i: (0, i, 0)),
            pl.BlockSpec((1, D), lambda i: (0, 0)),
            pl.BlockSpec((D, D), lambda i: (0, 0)),
        ],
        out_specs=pl.BlockSpec((BR, D), lambda i: (i, 0)),
        out_shape=jax.ShapeDtypeStruct((N, D), jnp.float32),
    )(s1, hs1, deg3, b1, w2)


def _tc_c_body(s2_ref, hs2_ref, deg_ref, b2_ref, batch_ref, lw_ref, lb_ref,
               out_ref, acc_s, acc_c):
    i = pl.program_id(0)

    @pl.when(i == 0)
    def _init():
        acc_s[...] = jnp.zeros((G, D), jnp.float32)
        acc_c[...] = jnp.zeros((G, D), jnp.float32)

    dgp = deg_ref[...]
    dinv = lax.rsqrt(dgp[0] + dgp[1] + 1.0)
    s = s2_ref[...]
    o2 = (s[0] + s[1] + hs2_ref[...]) * dinv + b2_ref[...]
    bb = batch_ref[0]
    gids = lax.broadcasted_iota(jnp.int32, (G, BR), 0)
    mb = (gids == bb).astype(jnp.float32)
    acc_s[...] += jnp.dot(mb, o2, preferred_element_type=jnp.float32)
    acc_c[...] += jnp.broadcast_to(
        jnp.sum(mb, axis=1, keepdims=True), (G, D))

    @pl.when(i == NB - 1)
    def _fin():
        hg = acc_s[...] / jnp.maximum(acc_c[...], 1.0)
        out_ref[...] = jnp.dot(hg, lw_ref[...],
                               preferred_element_type=jnp.float32) + lb_ref[...]


def _tc_c(s2, hs2, deg3, b2, batch_r, lin_w, lin_b):
    return pl.pallas_call(
        _tc_c_body,
        grid=(NB,),
        in_specs=[
            pl.BlockSpec((2, BR, D), lambda i: (0, i, 0)),
            pl.BlockSpec((BR, D), lambda i: (i, 0)),
            pl.BlockSpec((2, BR, 1), lambda i: (0, i, 0)),
            pl.BlockSpec((1, D), lambda i: (0, 0)),
            pl.BlockSpec((1, 1, BR), lambda i: (i, 0, 0)),
            pl.BlockSpec((D, 10), lambda i: (0, 0)),
            pl.BlockSpec((1, 10), lambda i: (0, 0)),
        ],
        out_specs=pl.BlockSpec((G, 10), lambda i: (0, 0)),
        out_shape=jax.ShapeDtypeStruct((G, 10), jnp.float32),
        scratch_shapes=[
            pltpu.VMEM((G, D), jnp.float32),
            pltpu.VMEM((G, D), jnp.float32),
        ],
        compiler_params=pltpu.CompilerParams(
            dimension_semantics=("arbitrary",)),
    )(s2, hs2, deg3, b2, batch_r, lin_w, lin_b)


def kernel(x, edge_index, batch, W1, b1, W2, b2, lin_W, lin_b):
    pad_src = jnp.arange(EPAD, dtype=jnp.int32) % N
    pad_dst = N + (jnp.arange(EPAD, dtype=jnp.int32) % (NPAD - N))
    src_p = jnp.concatenate([edge_index[0], pad_src]).reshape(NW, NCH, CH)
    dst_p = jnp.concatenate([edge_index[1], pad_dst]).reshape(NW, NCH, CH)
    comb = jnp.stack([src_p, dst_p], axis=2)
    degp = _degree(comb)
    deg3 = degp[:, :N].reshape(2, N, 1)
    hs1 = _tc_a(x, W1, deg3)
    s1 = _scatter(hs1, comb)
    hs2 = _tc_b(s1, hs1, deg3, b1.reshape(1, D), W2)
    s2 = _scatter(hs2, comb)
    out = _tc_c(s2, hs2, deg3, b2.reshape(1, D),
                batch.reshape(NB, 1, BR), lin_W, lin_b.reshape(1, 10))
    return out


# speedup vs baseline: 1.0245x; 1.0245x over previous
import functools

import jax
import jax.numpy as jnp
from jax import lax
from jax.experimental import pallas as pl
from jax.experimental.pallas import tpu as pltpu
from jax.experimental.pallas import tpu_sc as plsc

N = 10000
E = 320000
D = 128
G = 64
NW = 32
EPW = E // NW
CH = 128
EPWP = 10240
NCH = EPWP // CH
EPAD = NW * EPWP - E
NPAD = 10240
RPT = NPAD // 16
NB = 5
BR = N // NB


def _degree_body(comb, deg_out, idx_v, ones_v, zero_v, deg_sh, ssem):
    cid = lax.axis_index("c")
    sid = lax.axis_index("s")
    wid = sid * 2 + cid
    for i in range(8):
        ones_v[pl.ds(i * 16, 16)] = jnp.full((16,), 1.0, jnp.float32)
    for i in range(40):
        zero_v[pl.ds(i * 16, 16)] = jnp.zeros((16,), jnp.float32)
    pltpu.sync_copy(zero_v, deg_sh.at[pl.ds(sid * 640, 640)])
    pltpu.sync_copy(comb.at[wid], idx_v)
    plsc.subcore_barrier()

    def step(j, carry):
        @pl.when(j >= 8)
        def _drain():
            pltpu.make_async_copy(ones_v, deg_sh.at[idx_v.at[j - 8, 1]],
                                  ssem.at[lax.rem(j, 8)]).wait()

        pltpu.async_copy(ones_v, deg_sh.at[idx_v.at[j, 1]],
                         ssem.at[lax.rem(j, 8)], add=True)
        return carry

    lax.fori_loop(0, NCH, step, 0)
    for t in range(8):
        pltpu.make_async_copy(ones_v, deg_sh.at[idx_v.at[NCH - 8 + t, 1]],
                              ssem.at[(NCH - 8 + t) % 8]).wait()
    plsc.subcore_barrier()
    pltpu.sync_copy(deg_sh.at[pl.ds(sid * 640, 640)],
                    deg_out.at[cid, pl.ds(sid * 640, 640)])


_degree = functools.partial(
    pl.kernel,
    out_type=jax.ShapeDtypeStruct((2, NPAD), jnp.float32),
    mesh=plsc.VectorSubcoreMesh(core_axis_name="c", subcore_axis_name="s"),
    scratch_types=[
        pltpu.VMEM((NCH, 2, CH), jnp.int32),
        pltpu.VMEM((CH,), jnp.float32),
        pltpu.VMEM((640,), jnp.float32),
        pltpu.VMEM_SHARED((NPAD,), jnp.float32),
        pltpu.SemaphoreType.DMA((8,)),
    ],
)(_degree_body)


def _scatter_body(hs, comb, out, idx_v, rows_v, acc_sh, gsem, isem, ssem,
                  fsem):
    cid = lax.axis_index("c")
    sid = lax.axis_index("s")
    wid = sid * 2 + cid

    def zrow(r, carry):
        for c in range(8):
            rows_v[0, r, pl.ds(c * 16, 16)] = jnp.zeros((16,), jnp.float32)
        return carry

    lax.fori_loop(0, CH, zrow, 0)
    for k in range(5):
        pltpu.async_copy(rows_v.at[0],
                         acc_sh.at[pl.ds(sid * RPT + k * CH, CH)], fsem.at[k])
    pltpu.sync_copy(comb.at[wid, 0], idx_v.at[0])
    for k in range(5):
        pltpu.make_async_copy(rows_v.at[0],
                              acc_sh.at[pl.ds(sid * RPT + k * CH, CH)],
                              fsem.at[k]).wait()
    plsc.subcore_barrier()
    pltpu.async_copy(hs.at[idx_v.at[0, 0]], rows_v.at[0], gsem.at[0])
    pltpu.async_copy(comb.at[wid, 1], idx_v.at[1], isem.at[1])

    def step(j, carry):
        p = lax.rem(j, 2)
        pn = 1 - p
        q = lax.rem(j, 3)
        qn = lax.rem(j + 1, 3)

        @pl.when((j + 1 < NCH) & (j >= 1))
        def _row_free():
            pltpu.make_async_copy(rows_v.at[pn],
                                  acc_sh.at[idx_v.at[lax.rem(j + 2, 3), 1]],
                                  ssem.at[pn]).wait()

        @pl.when(j + 1 < NCH)
        def _next_gather():
            pltpu.make_async_copy(comb.at[wid, j + 1], idx_v.at[qn],
                                  isem.at[qn]).wait()
            pltpu.async_copy(hs.at[idx_v.at[qn, 0]], rows_v.at[pn],
                             gsem.at[pn])

        pltpu.make_async_copy(hs.at[idx_v.at[q, 0]], rows_v.at[p],
                              gsem.at[p]).wait()
        pltpu.async_copy(rows_v.at[p], acc_sh.at[idx_v.at[q, 1]],
                         ssem.at[p], add=True)

        @pl.when(j + 2 < NCH)
        def _next_idx():
            pltpu.async_copy(comb.at[wid, j + 2], idx_v.at[lax.rem(j + 2, 3)],
                             isem.at[lax.rem(j + 2, 3)])

        return carry

    lax.fori_loop(0, NCH, step, 0)
    pltpu.make_async_copy(rows_v.at[(NCH - 2) % 2],
                          acc_sh.at[idx_v.at[(NCH - 2) % 3, 1]],
                          ssem.at[(NCH - 2) % 2]).wait()
    pltpu.make_async_copy(rows_v.at[(NCH - 1) % 2],
                          acc_sh.at[idx_v.at[(NCH - 1) % 3, 1]],
                          ssem.at[(NCH - 1) % 2]).wait()
    plsc.subcore_barrier()
    for k in range(5):
        r0 = sid * RPT + k * 128
        pltpu.async_copy(acc_sh.at[pl.ds(r0, 128)],
                         out.at[cid, pl.ds(r0, 128)], fsem.at[k])
    for k in range(5):
        r0 = sid * RPT + k * 128
        pltpu.make_async_copy(acc_sh.at[pl.ds(r0, 128)],
                              out.at[cid, pl.ds(r0, 128)], fsem.at[k]).wait()


_scatter = functools.partial(
    pl.kernel,
    out_type=jax.ShapeDtypeStruct((2, NPAD, D), jnp.float32),
    mesh=plsc.VectorSubcoreMesh(core_axis_name="c", subcore_axis_name="s"),
    scratch_types=[
        pltpu.VMEM((3, 2, CH), jnp.int32),
        pltpu.VMEM((2, CH, D), jnp.float32),
        pltpu.VMEM_SHARED((NPAD, D), jnp.float32),
        pltpu.SemaphoreType.DMA((2,)),
        pltpu.SemaphoreType.DMA((3,)),
        pltpu.SemaphoreType.DMA((2,)),
        pltpu.SemaphoreType.DMA((5,)),
    ],
)(_scatter_body)


def _tc_a_body(x_ref, w1_ref, deg_ref, hs_ref):
    dgp = deg_ref[...]
    dinv = lax.rsqrt(dgp[0] + dgp[1] + 1.0)
    hs_ref[...] = jnp.dot(x_ref[...], w1_ref[...],
                          preferred_element_type=jnp.float32) * dinv


def _tc_a(x, w1, deg3):
    return pl.pallas_call(
        _tc_a_body,
        grid=(NB,),
        in_specs=[
            pl.BlockSpec((BR, D), lambda i: (i, 0)),
            pl.BlockSpec((D, D), lambda i: (0, 0)),
            pl.BlockSpec((2, BR, 1), lambda i: (0, i, 0)),
        ],
        out_specs=pl.BlockSpec((BR, D), lambda i: (i, 0)),
        out_shape=jax.ShapeDtypeStruct((N, D), jnp.float32),
    )(x, w1, deg3)


def _tc_b_body(s1_ref, hs1_ref, deg_ref, b1_ref, w2_ref, hs2_ref):
    dgp = deg_ref[...]
    dinv = lax.rsqrt(dgp[0] + dgp[1] + 1.0)
    s = s1_ref[...]
    o1 = (s[0] + s[1] + hs1_ref[...]) * dinv + b1_ref[...]
    r = jnp.maximum(o1, 0.0)
    hs2_ref[...] = jnp.dot(r, w2_ref[...],
                           preferred_element_type=jnp.float32) * dinv


def _tc_b(s1, hs1, deg3, b1, w2):
    return pl.pallas_call(
        _tc_b_body,
        grid=(NB,),
        in_specs=[
            pl.BlockSpec((2, BR, D), lambda i: (0, i, 0)),
            pl.BlockSpec((BR, D), lambda i: (i, 0)),
            pl.BlockSpec((2, BR, 1), lambda i: (0, i, 0)),
            pl.BlockSpec((1, D), lambda i: (0, 0)),
            pl.BlockSpec((D, D), lambda i: (0, 0)),
        ],
        out_specs=pl.BlockSpec((BR, D), lambda i: (i, 0)),
        out_shape=jax.ShapeDtypeStruct((N, D), jnp.float32),
    )(s1, hs1, deg3, b1, w2)


def _tc_c_body(s2_ref, hs2_ref, deg_ref, b2_ref, batch_ref, lw_ref, lb_ref,
               out_ref, acc_s, acc_c):
    i = pl.program_id(0)

    @pl.when(i == 0)
    def _init():
        acc_s[...] = jnp.zeros((G, D), jnp.float32)
        acc_c[...] = jnp.zeros((G, D), jnp.float32)

    dgp = deg_ref[...]
    dinv = lax.rsqrt(dgp[0] + dgp[1] + 1.0)
    s = s2_ref[...]
    o2 = (s[0] + s[1] + hs2_ref[...]) * dinv + b2_ref[...]
    bb = batch_ref[0]
    gids = lax.broadcasted_iota(jnp.int32, (G, BR), 0)
    mb = (gids == bb).astype(jnp.float32)
    acc_s[...] += jnp.dot(mb, o2, preferred_element_type=jnp.float32)
    acc_c[...] += jnp.broadcast_to(
        jnp.sum(mb, axis=1, keepdims=True), (G, D))

    @pl.when(i == NB - 1)
    def _fin():
        hg = acc_s[...] / jnp.maximum(acc_c[...], 1.0)
        out_ref[...] = jnp.dot(hg, lw_ref[...],
                               preferred_element_type=jnp.float32) + lb_ref[...]


def _tc_c(s2, hs2, deg3, b2, batch_r, lin_w, lin_b):
    return pl.pallas_call(
        _tc_c_body,
        grid=(NB,),
        in_specs=[
            pl.BlockSpec((2, BR, D), lambda i: (0, i, 0)),
            pl.BlockSpec((BR, D), lambda i: (i, 0)),
            pl.BlockSpec((2, BR, 1), lambda i: (0, i, 0)),
            pl.BlockSpec((1, D), lambda i: (0, 0)),
            pl.BlockSpec((1, 1, BR), lambda i: (i, 0, 0)),
            pl.BlockSpec((D, 10), lambda i: (0, 0)),
            pl.BlockSpec((1, 10), lambda i: (0, 0)),
        ],
        out_specs=pl.BlockSpec((G, 10), lambda i: (0, 0)),
        out_shape=jax.ShapeDtypeStruct((G, 10), jnp.float32),
        scratch_shapes=[
            pltpu.VMEM((G, D), jnp.float32),
            pltpu.VMEM((G, D), jnp.float32),
        ],
        compiler_params=pltpu.CompilerParams(
            dimension_semantics=("arbitrary",)),
    )(s2, hs2, deg3, b2, batch_r, lin_w, lin_b)


def kernel(x, edge_index, batch, W1, b1, W2, b2, lin_W, lin_b):
    pad_src = jnp.arange(EPAD, dtype=jnp.int32) % N
    pad_dst = N + (jnp.arange(EPAD, dtype=jnp.int32) % (NPAD - N))
    src_p = jnp.concatenate([edge_index[0], pad_src]).reshape(NW, NCH, CH)
    dst_p = jnp.concatenate([edge_index[1], pad_dst]).reshape(NW, NCH, CH)
    comb = jnp.stack([src_p, dst_p], axis=2)
    degp = _degree(comb)
    deg3 = degp[:, :N].reshape(2, N, 1)
    hs1 = _tc_a(x, W1, deg3)
    s1 = _scatter(hs1, comb)
    hs2 = _tc_b(s1, hs1, deg3, b1.reshape(1, D), W2)
    s2 = _scatter(hs2, comb)
    out = _tc_c(s2, hs2, deg3, b2.reshape(1, D),
                batch.reshape(NB, 1, BR), lin_W, lin_b.reshape(1, 10))
    return out


# speedup vs baseline: 1.0284x; 1.0039x over previous
import functools

import jax
import jax.numpy as jnp
from jax import lax
from jax.experimental import pallas as pl
from jax.experimental.pallas import tpu as pltpu
from jax.experimental.pallas import tpu_sc as plsc

N = 10000
E = 320000
D = 128
G = 64
NW = 32
EPW = E // NW
CH = 128
EPWP = 10240
NCH = EPWP // CH
EPAD = NW * EPWP - E
NPAD = 10240
RPT = NPAD // 16
NB = 2
BR = N // NB


def _degree_body(comb, deg_out, idx_v, ones_v, zero_v, deg_sh, ssem):
    cid = lax.axis_index("c")
    sid = lax.axis_index("s")
    wid = sid * 2 + cid
    for i in range(8):
        ones_v[pl.ds(i * 16, 16)] = jnp.full((16,), 1.0, jnp.float32)
    for i in range(40):
        zero_v[pl.ds(i * 16, 16)] = jnp.zeros((16,), jnp.float32)
    pltpu.sync_copy(zero_v, deg_sh.at[pl.ds(sid * 640, 640)])
    pltpu.sync_copy(comb.at[wid], idx_v)
    plsc.subcore_barrier()

    def step(j, carry):
        @pl.when(j >= 8)
        def _drain():
            pltpu.make_async_copy(ones_v, deg_sh.at[idx_v.at[j - 8, 1]],
                                  ssem.at[lax.rem(j, 8)]).wait()

        pltpu.async_copy(ones_v, deg_sh.at[idx_v.at[j, 1]],
                         ssem.at[lax.rem(j, 8)], add=True)
        return carry

    lax.fori_loop(0, NCH, step, 0)
    for t in range(8):
        pltpu.make_async_copy(ones_v, deg_sh.at[idx_v.at[NCH - 8 + t, 1]],
                              ssem.at[(NCH - 8 + t) % 8]).wait()
    plsc.subcore_barrier()
    pltpu.sync_copy(deg_sh.at[pl.ds(sid * 640, 640)],
                    deg_out.at[cid, pl.ds(sid * 640, 640)])


_degree = functools.partial(
    pl.kernel,
    out_type=jax.ShapeDtypeStruct((2, NPAD), jnp.float32),
    mesh=plsc.VectorSubcoreMesh(core_axis_name="c", subcore_axis_name="s"),
    scratch_types=[
        pltpu.VMEM((NCH, 2, CH), jnp.int32),
        pltpu.VMEM((CH,), jnp.float32),
        pltpu.VMEM((640,), jnp.float32),
        pltpu.VMEM_SHARED((NPAD,), jnp.float32),
        pltpu.SemaphoreType.DMA((8,)),
    ],
)(_degree_body)


def _scatter_body(hs, comb, out, idx_v, rows_v, acc_sh, gsem, isem, ssem,
                  fsem):
    cid = lax.axis_index("c")
    sid = lax.axis_index("s")
    wid = sid * 2 + cid

    def zrow(r, carry):
        for c in range(8):
            rows_v[0, r, pl.ds(c * 16, 16)] = jnp.zeros((16,), jnp.float32)
        return carry

    lax.fori_loop(0, CH, zrow, 0)
    for k in range(5):
        pltpu.async_copy(rows_v.at[0],
                         acc_sh.at[pl.ds(sid * RPT + k * CH, CH)], fsem.at[k])
    pltpu.sync_copy(comb.at[wid, 0], idx_v.at[0])
    for k in range(5):
        pltpu.make_async_copy(rows_v.at[0],
                              acc_sh.at[pl.ds(sid * RPT + k * CH, CH)],
                              fsem.at[k]).wait()
    plsc.subcore_barrier()
    pltpu.async_copy(hs.at[idx_v.at[0, 0]], rows_v.at[0], gsem.at[0])
    pltpu.async_copy(comb.at[wid, 1], idx_v.at[1], isem.at[1])

    def step(j, carry):
        p = lax.rem(j, 2)
        pn = 1 - p
        q = lax.rem(j, 3)
        qn = lax.rem(j + 1, 3)

        @pl.when((j + 1 < NCH) & (j >= 1))
        def _row_free():
            pltpu.make_async_copy(rows_v.at[pn],
                                  acc_sh.at[idx_v.at[lax.rem(j + 2, 3), 1]],
                                  ssem.at[pn]).wait()

        @pl.when(j + 1 < NCH)
        def _next_gather():
            pltpu.make_async_copy(comb.at[wid, j + 1], idx_v.at[qn],
                                  isem.at[qn]).wait()
            pltpu.async_copy(hs.at[idx_v.at[qn, 0]], rows_v.at[pn],
                             gsem.at[pn])

        pltpu.make_async_copy(hs.at[idx_v.at[q, 0]], rows_v.at[p],
                              gsem.at[p]).wait()
        pltpu.async_copy(rows_v.at[p], acc_sh.at[idx_v.at[q, 1]],
                         ssem.at[p], add=True)

        @pl.when(j + 2 < NCH)
        def _next_idx():
            pltpu.async_copy(comb.at[wid, j + 2], idx_v.at[lax.rem(j + 2, 3)],
                             isem.at[lax.rem(j + 2, 3)])

        return carry

    lax.fori_loop(0, NCH, step, 0)
    pltpu.make_async_copy(rows_v.at[(NCH - 2) % 2],
                          acc_sh.at[idx_v.at[(NCH - 2) % 3, 1]],
                          ssem.at[(NCH - 2) % 2]).wait()
    pltpu.make_async_copy(rows_v.at[(NCH - 1) % 2],
                          acc_sh.at[idx_v.at[(NCH - 1) % 3, 1]],
                          ssem.at[(NCH - 1) % 2]).wait()
    plsc.subcore_barrier()
    for k in range(5):
        r0 = sid * RPT + k * 128
        pltpu.async_copy(acc_sh.at[pl.ds(r0, 128)],
                         out.at[cid, pl.ds(r0, 128)], fsem.at[k])
    for k in range(5):
        r0 = sid * RPT + k * 128
        pltpu.make_async_copy(acc_sh.at[pl.ds(r0, 128)],
                              out.at[cid, pl.ds(r0, 128)], fsem.at[k]).wait()


_scatter = functools.partial(
    pl.kernel,
    out_type=jax.ShapeDtypeStruct((2, NPAD, D), jnp.float32),
    mesh=plsc.VectorSubcoreMesh(core_axis_name="c", subcore_axis_name="s"),
    scratch_types=[
        pltpu.VMEM((3, 2, CH), jnp.int32),
        pltpu.VMEM((2, CH, D), jnp.float32),
        pltpu.VMEM_SHARED((NPAD, D), jnp.float32),
        pltpu.SemaphoreType.DMA((2,)),
        pltpu.SemaphoreType.DMA((3,)),
        pltpu.SemaphoreType.DMA((2,)),
        pltpu.SemaphoreType.DMA((5,)),
    ],
)(_scatter_body)


def _tc_a_body(x_ref, w1_ref, deg_ref, hs_ref):
    dgp = deg_ref[...]
    dinv = lax.rsqrt(dgp[0] + dgp[1] + 1.0)
    hs_ref[...] = jnp.dot(x_ref[...], w1_ref[...],
                          preferred_element_type=jnp.float32) * dinv


def _tc_a(x, w1, deg3):
    return pl.pallas_call(
        _tc_a_body,
        grid=(NB,),
        in_specs=[
            pl.BlockSpec((BR, D), lambda i: (i, 0)),
            pl.BlockSpec((D, D), lambda i: (0, 0)),
            pl.BlockSpec((2, BR, 1), lambda i: (0, i, 0)),
        ],
        out_specs=pl.BlockSpec((BR, D), lambda i: (i, 0)),
        out_shape=jax.ShapeDtypeStruct((N, D), jnp.float32),
    )(x, w1, deg3)


def _tc_b_body(s1_ref, hs1_ref, deg_ref, b1_ref, w2_ref, hs2_ref):
    dgp = deg_ref[...]
    dinv = lax.rsqrt(dgp[0] + dgp[1] + 1.0)
    s = s1_ref[...]
    o1 = (s[0] + s[1] + hs1_ref[...]) * dinv + b1_ref[...]
    r = jnp.maximum(o1, 0.0)
    hs2_ref[...] = jnp.dot(r, w2_ref[...],
                           preferred_element_type=jnp.float32) * dinv


def _tc_b(s1, hs1, deg3, b1, w2):
    return pl.pallas_call(
        _tc_b_body,
        grid=(NB,),
        in_specs=[
            pl.BlockSpec((2, BR, D), lambda i: (0, i, 0)),
            pl.BlockSpec((BR, D), lambda i: (i, 0)),
            pl.BlockSpec((2, BR, 1), lambda i: (0, i, 0)),
            pl.BlockSpec((1, D), lambda i: (0, 0)),
            pl.BlockSpec((D, D), lambda i: (0, 0)),
        ],
        out_specs=pl.BlockSpec((BR, D), lambda i: (i, 0)),
        out_shape=jax.ShapeDtypeStruct((N, D), jnp.float32),
    )(s1, hs1, deg3, b1, w2)


def _tc_c_body(s2_ref, hs2_ref, deg_ref, b2_ref, batch_ref, lw_ref, lb_ref,
               out_ref, acc_s, acc_c):
    i = pl.program_id(0)

    @pl.when(i == 0)
    def _init():
        acc_s[...] = jnp.zeros((G, D), jnp.float32)
        acc_c[...] = jnp.zeros((G, D), jnp.float32)

    dgp = deg_ref[...]
    dinv = lax.rsqrt(dgp[0] + dgp[1] + 1.0)
    s = s2_ref[...]
    o2 = (s[0] + s[1] + hs2_ref[...]) * dinv + b2_ref[...]
    bb = batch_ref[0]
    gids = lax.broadcasted_iota(jnp.int32, (G, BR), 0)
    mb = (gids == bb).astype(jnp.float32)
    acc_s[...] += jnp.dot(mb, o2, preferred_element_type=jnp.float32)
    acc_c[...] += jnp.broadcast_to(
        jnp.sum(mb, axis=1, keepdims=True), (G, D))

    @pl.when(i == NB - 1)
    def _fin():
        hg = acc_s[...] / jnp.maximum(acc_c[...], 1.0)
        out_ref[...] = jnp.dot(hg, lw_ref[...],
                               preferred_element_type=jnp.float32) + lb_ref[...]


def _tc_c(s2, hs2, deg3, b2, batch_r, lin_w, lin_b):
    return pl.pallas_call(
        _tc_c_body,
        grid=(NB,),
        in_specs=[
            pl.BlockSpec((2, BR, D), lambda i: (0, i, 0)),
            pl.BlockSpec((BR, D), lambda i: (i, 0)),
            pl.BlockSpec((2, BR, 1), lambda i: (0, i, 0)),
            pl.BlockSpec((1, D), lambda i: (0, 0)),
            pl.BlockSpec((1, 1, BR), lambda i: (i, 0, 0)),
            pl.BlockSpec((D, 10), lambda i: (0, 0)),
            pl.BlockSpec((1, 10), lambda i: (0, 0)),
        ],
        out_specs=pl.BlockSpec((G, 10), lambda i: (0, 0)),
        out_shape=jax.ShapeDtypeStruct((G, 10), jnp.float32),
        scratch_shapes=[
            pltpu.VMEM((G, D), jnp.float32),
            pltpu.VMEM((G, D), jnp.float32),
        ],
        compiler_params=pltpu.CompilerParams(
            dimension_semantics=("arbitrary",)),
    )(s2, hs2, deg3, b2, batch_r, lin_w, lin_b)


def kernel(x, edge_index, batch, W1, b1, W2, b2, lin_W, lin_b):
    pad_src = jnp.arange(EPAD, dtype=jnp.int32) % N
    pad_dst = N + (jnp.arange(EPAD, dtype=jnp.int32) % (NPAD - N))
    src_p = jnp.concatenate([edge_index[0], pad_src]).reshape(NW, NCH, CH)
    dst_p = jnp.concatenate([edge_index[1], pad_dst]).reshape(NW, NCH, CH)
    comb = jnp.stack([src_p, dst_p], axis=2)
    degp = _degree(comb)
    deg3 = degp[:, :N].reshape(2, N, 1)
    hs1 = _tc_a(x, W1, deg3)
    s1 = _scatter(hs1, comb)
    hs2 = _tc_b(s1, hs1, deg3, b1.reshape(1, D), W2)
    s2 = _scatter(hs2, comb)
    out = _tc_c(s2, hs2, deg3, b2.reshape(1, D),
                batch.reshape(NB, 1, BR), lin_W, lin_b.reshape(1, 10))
    return out
